# baseline (speedup 1.0000x reference)
import jax
import jax.numpy as jnp
from jax import lax
from jax.experimental import pallas as pl
from jax.experimental.pallas import tpu as pltpu
from jax.experimental.pallas import tpu_sc as plsc

E = 64
TOP_K = 2
D_MODEL = 1024
D_FF = 1024
T = 512
BM = 128
NP = T * TOP_K
MS = NP + 7 * E + BM

_NEG = -3.4e38


def _routing_tables(x_ref, gw_ref, off_ref, ntk_ref, pos_ref, wbrd_ref):
    x = x_ref[...]
    gw = gw_ref[...]
    logits = lax.dot_general(
        x, gw, (((1,), (1,)), ((), ())), preferred_element_type=jnp.float32
    )
    m1 = jnp.max(logits, axis=1, keepdims=True)
    s1 = logits >= m1
    m2 = jnp.max(jnp.where(s1, _NEG, logits), axis=1, keepdims=True)
    s2 = (logits >= m2) & (~s1)
    w1 = 1.0 / (1.0 + jnp.exp(m2 - m1))
    w2 = 1.0 - w1
    onht = jnp.concatenate(
        [jnp.where(s1, 1.0, 0.0), jnp.where(s2, 1.0, 0.0)], axis=0
    )
    hist = lax.dot_general(
        jnp.ones((1, NP), jnp.float32), onht, (((1,), (0,)), ((), ())),
        preferred_element_type=jnp.float32,
    )
    hist8 = jnp.floor((hist + 7.0) * 0.125) * 8.0
    ue = lax.broadcasted_iota(jnp.int32, (E, E), 0) < lax.broadcasted_iota(
        jnp.int32, (E, E), 1
    )
    off8 = lax.dot_general(
        hist8, jnp.where(ue, 1.0, 0.0), (((1,), (0,)), ((), ())),
        preferred_element_type=jnp.float32,
    )
    up = lax.broadcasted_iota(jnp.int32, (NP, NP), 0) > lax.broadcasted_iota(
        jnp.int32, (NP, NP), 1
    )
    prev = lax.dot_general(
        jnp.where(up, 1.0, 0.0), onht, (((1,), (0,)), ((), ())),
        preferred_element_type=jnp.float32,
    )
    rank = jnp.sum(onht * prev, axis=1, keepdims=True)
    base = jnp.sum(onht * off8, axis=1, keepdims=True)
    pos_ref[...] = (base + rank).astype(jnp.int32)
    off_ref[...] = off8.astype(jnp.int32)
    ntk_ref[...] = hist.astype(jnp.int32)
    wsel = jnp.concatenate([w1, w2], axis=0)
    wbrd_ref[...] = jnp.broadcast_to(wsel, (NP, 128))


def _dispatch_body(x_hbm, wbrd_hbm, pos_hbm, xs_out, ws_out, idx_v, xb_v, wb_v, semx, semw):
    wid = lax.axis_index("s") * 2 + lax.axis_index("c")
    base = wid * 32
    pltpu.sync_copy(pos_hbm.at[pl.ds(base, 32)], idx_v)
    src = lax.rem(base, T)
    pltpu.sync_copy(x_hbm.at[pl.ds(src, 32)], xb_v)
    pltpu.sync_copy(wbrd_hbm.at[pl.ds(base, 32)], wb_v)
    pltpu.async_copy(xb_v, xs_out.at[idx_v], semx).wait()
    pltpu.async_copy(wb_v, ws_out.at[idx_v], semw).wait()


def _ffn_body(off_ref, ntk_ref, xs_ref, ws_ref, gu_ref, dn_ref, y_ref):
    e = pl.program_id(0)
    off = off_ref[e]
    n = ntk_ref[e]
    wgu = gu_ref[0]
    wd = dn_ref[0]
    for b in range(T // BM):

        @pl.when(n > BM * b)
        def _block(b=b):
            row = pl.multiple_of(off + BM * b, 8)
            xb = xs_ref[pl.ds(row, BM), :]
            gu = lax.dot_general(
                xb, wgu, (((1,), (1,)), ((), ())), preferred_element_type=jnp.float32
            )
            g = gu[:, :D_FF]
            u = gu[:, D_FF:]
            h = g / (1.0 + jnp.exp(-g)) * u
            eo = lax.dot_general(
                h, wd, (((1,), (1,)), ((), ())), preferred_element_type=jnp.float32
            )
            w = ws_ref[pl.ds(row, BM), :][:, 0:1]
            y_ref[pl.ds(row, BM), :] = eo * w


def _combine_body(y_hbm, pos_hbm, out_hbm, idx0_v, idx1_v, y0_v, y1_v, sem0, sem1):
    wid = lax.axis_index("s") * 2 + lax.axis_index("c")
    t0 = wid * 16
    pltpu.sync_copy(pos_hbm.at[pl.ds(t0, 16)], idx0_v)
    pltpu.sync_copy(pos_hbm.at[pl.ds(T + t0, 16)], idx1_v)
    pltpu.async_copy(y_hbm.at[idx0_v], y0_v, sem0).wait()
    pltpu.async_copy(y_hbm.at[idx1_v], y1_v, sem1).wait()

    def _row(r, carry):
        def _col(cc, carry2):
            for k in range(4):
                sl = pl.ds((cc * 4 + k) * 16, 16)
                y0_v[r, sl] = y0_v[r, sl] + y1_v[r, sl]
            return carry2

        return lax.fori_loop(0, 16, _col, carry)

    lax.fori_loop(0, 16, _row, 0)
    pltpu.sync_copy(y0_v, out_hbm.at[pl.ds(t0, 16)])


def kernel(hidden_states, gate_weight, gate_up_proj, down_proj):
    off2d, ntk2d, pos2d, wbrd = pl.pallas_call(
        _routing_tables,
        out_shape=[
            jax.ShapeDtypeStruct((1, E), jnp.int32),
            jax.ShapeDtypeStruct((1, E), jnp.int32),
            jax.ShapeDtypeStruct((NP, 1), jnp.int32),
            jax.ShapeDtypeStruct((NP, 128), jnp.float32),
        ],
    )(hidden_states, gate_weight)
    off8 = off2d.reshape(E)
    ntk = ntk2d.reshape(E)
    pos = pos2d.reshape(NP)

    mesh = plsc.VectorSubcoreMesh(
        core_axis_name="c", subcore_axis_name="s", num_cores=2, num_subcores=16
    )
    xs, ws = pl.kernel(
        _dispatch_body,
        out_type=(
            jax.ShapeDtypeStruct((MS, D_MODEL), jnp.float32),
            jax.ShapeDtypeStruct((MS, 128), jnp.float32),
        ),
        mesh=mesh,
        scratch_types=[
            pltpu.VMEM((32,), jnp.int32),
            pltpu.VMEM((32, D_MODEL), jnp.float32),
            pltpu.VMEM((32, 128), jnp.float32),
            pltpu.SemaphoreType.DMA,
            pltpu.SemaphoreType.DMA,
        ],
    )(hidden_states, wbrd, pos)

    y = pl.pallas_call(
        _ffn_body,
        grid_spec=pltpu.PrefetchScalarGridSpec(
            num_scalar_prefetch=2,
            grid=(E,),
            in_specs=[
                pl.BlockSpec((MS, D_MODEL), lambda e, o, nt: (0, 0)),
                pl.BlockSpec((MS, 128), lambda e, o, nt: (0, 0)),
                pl.BlockSpec((1, 2 * D_FF, D_MODEL), lambda e, o, nt: (e, 0, 0)),
                pl.BlockSpec((1, D_MODEL, D_FF), lambda e, o, nt: (e, 0, 0)),
            ],
            out_specs=pl.BlockSpec((MS, D_MODEL), lambda e, o, nt: (0, 0)),
        ),
        out_shape=jax.ShapeDtypeStruct((MS, D_MODEL), jnp.float32),
        compiler_params=pltpu.CompilerParams(
            dimension_semantics=("arbitrary",),
        ),
    )(off8, ntk, xs, ws, gate_up_proj, down_proj)

    out = pl.kernel(
        _combine_body,
        out_type=jax.ShapeDtypeStruct((T, D_MODEL), jnp.float32),
        mesh=mesh,
        scratch_types=[
            pltpu.VMEM((16,), jnp.int32),
            pltpu.VMEM((16,), jnp.int32),
            pltpu.VMEM((16, D_MODEL), jnp.float32),
            pltpu.VMEM((16, D_MODEL), jnp.float32),
            pltpu.SemaphoreType.DMA,
            pltpu.SemaphoreType.DMA,
        ],
    )(y, pos)
    return out


# speedup vs baseline: 1.1346x; 1.1346x over previous
import jax
import jax.numpy as jnp
from jax import lax
from jax.experimental import pallas as pl
from jax.experimental.pallas import tpu as pltpu

E = 64
TOP_K = 2
D_MODEL = 1024
D_FF = 1024
T = 512
BM = 64

_NEG = -3.4e38


def _moe_body(x_ref, gw_ref, gu_ref, dn_ref, out_ref, st_ref, rt_ref, gt_ref, acc_ref):
    e = pl.program_id(0)

    @pl.when(e == 0)
    def _routing():
        x = x_ref[...]
        gw = gw_ref[...]
        logits_t = lax.dot_general(
            gw, x, (((1,), (1,)), ((), ())), preferred_element_type=jnp.float32
        )
        m1 = jnp.max(logits_t, axis=0, keepdims=True)
        s1 = logits_t >= m1
        masked = jnp.where(s1, _NEG, logits_t)
        m2 = jnp.max(masked, axis=0, keepdims=True)
        s2 = (logits_t >= m2) & (~s1)
        w1 = 1.0 / (1.0 + jnp.exp(m2 - m1))
        w2 = 1.0 - w1
        occ = jnp.where(s1 | s2, 1.0, 0.0)
        r_iota = lax.broadcasted_iota(jnp.int32, (T, T), 0)
        c_iota = lax.broadcasted_iota(jnp.int32, (T, T), 1)
        upper = jnp.where(r_iota < c_iota, 1.0, 0.0)
        rt = lax.dot_general(
            occ, upper, (((1,), (0,)), ((), ())), preferred_element_type=jnp.float32
        )
        st_ref[...] = occ
        rt_ref[...] = rt
        gt_ref[...] = jnp.where(s1, w1, 0.0) + jnp.where(s2, w2, 0.0)
        acc_ref[...] = jnp.zeros((T, D_MODEL), jnp.float32)

    srow = st_ref[pl.ds(e, 1), :]
    rrow = rt_ref[pl.ds(e, 1), :]
    grow = gt_ref[pl.ds(e, 1), :]
    n = jnp.sum(srow)
    wgu = gu_ref[0]
    wd = dn_ref[0]

    for b in range(T // BM):

        @pl.when(n > float(BM * b))
        def _block(b=b):
            rr = lax.broadcasted_iota(jnp.int32, (BM, T), 0).astype(
                jnp.float32
            ) + float(BM * b)
            sel = jnp.where((rrow == rr) & (srow > 0.0), 1.0, 0.0)
            xb = lax.dot_general(
                sel, x_ref[...], (((1,), (0,)), ((), ())),
                preferred_element_type=jnp.float32,
            )
            gu = lax.dot_general(
                xb, wgu, (((1,), (1,)), ((), ())), preferred_element_type=jnp.float32
            )
            g = gu[:, :D_FF]
            u = gu[:, D_FF:]
            h = g / (1.0 + jnp.exp(-g)) * u
            eo = lax.dot_general(
                h, wd, (((1,), (1,)), ((), ())), preferred_element_type=jnp.float32
            )
            wcol = lax.dot_general(
                sel, grow, (((1,), (1,)), ((), ())), preferred_element_type=jnp.float32
            )
            acc_ref[...] += lax.dot_general(
                sel, eo * wcol, (((0,), (0,)), ((), ())),
                preferred_element_type=jnp.float32,
            )

    @pl.when(e == E - 1)
    def _emit():
        out_ref[...] = acc_ref[...]


def kernel(hidden_states, gate_weight, gate_up_proj, down_proj):
    return pl.pallas_call(
        _moe_body,
        grid=(E,),
        in_specs=[
            pl.BlockSpec((T, D_MODEL), lambda e: (0, 0)),
            pl.BlockSpec((E, D_MODEL), lambda e: (0, 0)),
            pl.BlockSpec((1, 2 * D_FF, D_MODEL), lambda e: (e, 0, 0)),
            pl.BlockSpec((1, D_MODEL, D_FF), lambda e: (e, 0, 0)),
        ],
        out_specs=pl.BlockSpec((T, D_MODEL), lambda e: (0, 0)),
        out_shape=jax.ShapeDtypeStruct((T, D_MODEL), jnp.float32),
        scratch_shapes=[
            pltpu.VMEM((E, T), jnp.float32),
            pltpu.VMEM((E, T), jnp.float32),
            pltpu.VMEM((E, T), jnp.float32),
            pltpu.VMEM((T, D_MODEL), jnp.float32),
        ],
        compiler_params=pltpu.CompilerParams(
            dimension_semantics=("arbitrary",),
        ),
    )(hidden_states, gate_weight, gate_up_proj, down_proj)


# speedup vs baseline: 1.1422x; 1.0067x over previous
import jax
import jax.numpy as jnp
from jax import lax
from jax.experimental import pallas as pl
from jax.experimental.pallas import tpu as pltpu

E = 64
TOP_K = 2
D_MODEL = 1024
D_FF = 1024
T = 512
BM = 32

_NEG = -3.4e38


def _moe_body(x_ref, gw_ref, gu_ref, dn_ref, out_ref, st_ref, rt_ref, gt_ref, acc_ref):
    e = pl.program_id(0)

    @pl.when(e == 0)
    def _routing():
        x = x_ref[...]
        gw = gw_ref[...]
        logits_t = lax.dot_general(
            gw, x, (((1,), (1,)), ((), ())), preferred_element_type=jnp.float32
        )
        m1 = jnp.max(logits_t, axis=0, keepdims=True)
        s1 = logits_t >= m1
        masked = jnp.where(s1, _NEG, logits_t)
        m2 = jnp.max(masked, axis=0, keepdims=True)
        s2 = (logits_t >= m2) & (~s1)
        w1 = 1.0 / (1.0 + jnp.exp(m2 - m1))
        w2 = 1.0 - w1
        occ = jnp.where(s1 | s2, 1.0, 0.0)
        r_iota = lax.broadcasted_iota(jnp.int32, (T, T), 0)
        c_iota = lax.broadcasted_iota(jnp.int32, (T, T), 1)
        upper = jnp.where(r_iota < c_iota, 1.0, 0.0)
        rt = lax.dot_general(
            occ, upper, (((1,), (0,)), ((), ())), preferred_element_type=jnp.float32
        )
        st_ref[...] = occ
        rt_ref[...] = rt
        gt_ref[...] = jnp.where(s1, w1, 0.0) + jnp.where(s2, w2, 0.0)
        acc_ref[...] = jnp.zeros((T, D_MODEL), jnp.float32)

    srow = st_ref[pl.ds(e, 1), :]
    rrow = rt_ref[pl.ds(e, 1), :]
    grow = gt_ref[pl.ds(e, 1), :]
    n = jnp.sum(srow)
    wgu = gu_ref[0]
    wd = dn_ref[0]

    for b in range(T // BM):

        @pl.when(n > float(BM * b))
        def _block(b=b):
            rr = lax.broadcasted_iota(jnp.int32, (BM, T), 0).astype(
                jnp.float32
            ) + float(BM * b)
            sel = jnp.where((rrow == rr) & (srow > 0.0), 1.0, 0.0)
            xb = lax.dot_general(
                sel, x_ref[...], (((1,), (0,)), ((), ())),
                preferred_element_type=jnp.float32,
            )
            gu = lax.dot_general(
                xb, wgu, (((1,), (1,)), ((), ())), preferred_element_type=jnp.float32
            )
            g = gu[:, :D_FF]
            u = gu[:, D_FF:]
            h = g / (1.0 + jnp.exp(-g)) * u
            eo = lax.dot_general(
                h, wd, (((1,), (1,)), ((), ())), preferred_element_type=jnp.float32
            )
            wcol = lax.dot_general(
                sel, grow, (((1,), (1,)), ((), ())), preferred_element_type=jnp.float32
            )
            acc_ref[...] += lax.dot_general(
                sel, eo * wcol, (((0,), (0,)), ((), ())),
                preferred_element_type=jnp.float32,
            )

    @pl.when(e == E - 1)
    def _emit():
        out_ref[...] = acc_ref[...]


def kernel(hidden_states, gate_weight, gate_up_proj, down_proj):
    return pl.pallas_call(
        _moe_body,
        grid=(E,),
        in_specs=[
            pl.BlockSpec((T, D_MODEL), lambda e: (0, 0)),
            pl.BlockSpec((E, D_MODEL), lambda e: (0, 0)),
            pl.BlockSpec((1, 2 * D_FF, D_MODEL), lambda e: (e, 0, 0)),
            pl.BlockSpec((1, D_MODEL, D_FF), lambda e: (e, 0, 0)),
        ],
        out_specs=pl.BlockSpec((T, D_MODEL), lambda e: (0, 0)),
        out_shape=jax.ShapeDtypeStruct((T, D_MODEL), jnp.float32),
        scratch_shapes=[
            pltpu.VMEM((E, T), jnp.float32),
            pltpu.VMEM((E, T), jnp.float32),
            pltpu.VMEM((E, T), jnp.float32),
            pltpu.VMEM((T, D_MODEL), jnp.float32),
        ],
        compiler_params=pltpu.CompilerParams(
            dimension_semantics=("arbitrary",),
        ),
    )(hidden_states, gate_weight, gate_up_proj, down_proj)


# speedup vs baseline: 1.1809x; 1.0339x over previous
import jax
import jax.numpy as jnp
from jax import lax
from jax.experimental import pallas as pl
from jax.experimental.pallas import tpu as pltpu

E = 64
TOP_K = 2
D_MODEL = 1024
D_FF = 1024
T = 512
BM = 32

_NEG = -3.4e38


def _moe_body(
    x_ref, gw_ref, gu_ref, dn_ref, out_ref, st_ref, rt_ref, gt_ref, acc_ref,
    y_ref, cmb_ref,
):
    e = pl.program_id(0)

    @pl.when(e == 0)
    def _routing():
        x = x_ref[...]
        gw = gw_ref[...]
        logits_t = lax.dot_general(
            gw, x, (((1,), (1,)), ((), ())), preferred_element_type=jnp.float32
        )
        m1 = jnp.max(logits_t, axis=0, keepdims=True)
        s1 = logits_t >= m1
        masked = jnp.where(s1, _NEG, logits_t)
        m2 = jnp.max(masked, axis=0, keepdims=True)
        s2 = (logits_t >= m2) & (~s1)
        w1 = 1.0 / (1.0 + jnp.exp(m2 - m1))
        w2 = 1.0 - w1
        occ = jnp.where(s1 | s2, 1.0, 0.0)
        r_iota = lax.broadcasted_iota(jnp.int32, (T, T), 0)
        c_iota = lax.broadcasted_iota(jnp.int32, (T, T), 1)
        upper = jnp.where(r_iota < c_iota, 1.0, 0.0)
        rt = lax.dot_general(
            occ, upper, (((1,), (0,)), ((), ())), preferred_element_type=jnp.float32
        )
        st_ref[...] = occ
        rt_ref[...] = rt
        gt_ref[...] = jnp.where(s1, w1, 0.0) + jnp.where(s2, w2, 0.0)
        acc_ref[...] = jnp.zeros((T, D_MODEL), jnp.float32)
        y_ref[...] = jnp.zeros((E * BM, D_MODEL), jnp.float32)
        s1f = jnp.where(s1, 1.0, 0.0)
        s2f = jnp.where(s2, 1.0, 0.0)
        e_col = lax.broadcasted_iota(jnp.int32, (E, 1), 0).astype(jnp.float32)
        e0 = jnp.sum(s1f * e_col, axis=0, keepdims=True)
        e1 = jnp.sum(s2f * e_col, axis=0, keepdims=True)
        rank0 = jnp.sum(s1f * rt, axis=0, keepdims=True)
        rank1 = jnp.sum(s2f * rt, axis=0, keepdims=True)
        cmb_ref[0:1, :] = float(BM) * e0 + rank0
        cmb_ref[1:2, :] = float(BM) * e1 + rank1
        cmb_ref[2:3, :] = w1
        cmb_ref[3:4, :] = w2
        cmb_ref[4:5, :] = rank0
        cmb_ref[5:6, :] = rank1
        cmb_ref[6:8, :] = jnp.zeros((2, T), jnp.float32)

    srow = st_ref[pl.ds(e, 1), :]
    rrow = rt_ref[pl.ds(e, 1), :]
    n = jnp.sum(srow)
    wgu = gu_ref[0]
    wd = dn_ref[0]

    for b in range(T // BM):

        @pl.when(n > float(BM * b))
        def _block(b=b):
            rr = lax.broadcasted_iota(jnp.int32, (BM, T), 0).astype(
                jnp.float32
            ) + float(BM * b)
            sel = jnp.where((rrow == rr) & (srow > 0.0), 1.0, 0.0)
            xb = lax.dot_general(
                sel, x_ref[...], (((1,), (0,)), ((), ())),
                preferred_element_type=jnp.float32,
            )
            gu = lax.dot_general(
                xb, wgu, (((1,), (1,)), ((), ())), preferred_element_type=jnp.float32
            )
            g = gu[:, :D_FF]
            u = gu[:, D_FF:]
            h = g / (1.0 + jnp.exp(-g)) * u
            eo = lax.dot_general(
                h, wd, (((1,), (1,)), ((), ())), preferred_element_type=jnp.float32
            )
            if b == 0:
                y_ref[pl.ds(pl.multiple_of(e * BM, BM), BM), :] = eo
            else:
                grow = gt_ref[pl.ds(e, 1), :]
                wcol = lax.dot_general(
                    sel, grow, (((1,), (1,)), ((), ())),
                    preferred_element_type=jnp.float32,
                )
                acc_ref[...] += lax.dot_general(
                    sel, eo * wcol, (((0,), (0,)), ((), ())),
                    preferred_element_type=jnp.float32,
                )

    @pl.when(e == E - 1)
    def _emit():
        scr = cmb_ref[...]
        r_i = lax.broadcasted_iota(jnp.int32, (T, T), 0)
        c_i = lax.broadcasted_iota(jnp.int32, (T, T), 1)
        ident = jnp.where(r_i == c_i, 1.0, 0.0)
        scr_t = lax.dot_general(
            ident, scr, (((1,), (1,)), ((), ())), preferred_element_type=jnp.float32
        )
        slot0 = scr_t[:, 0:1]
        slot1 = scr_t[:, 1:2]
        w1c = scr_t[:, 2:3]
        w2c = scr_t[:, 3:4]
        rk0 = scr_t[:, 4:5]
        rk1 = scr_t[:, 5:6]
        slots = lax.broadcasted_iota(jnp.int32, (T, E * BM), 1).astype(jnp.float32)
        oh = jnp.where((slots == slot0) & (rk0 < float(BM)), w1c, 0.0) + jnp.where(
            (slots == slot1) & (rk1 < float(BM)), w2c, 0.0
        )
        out_ref[...] = acc_ref[...] + lax.dot_general(
            oh, y_ref[...], (((1,), (0,)), ((), ())),
            preferred_element_type=jnp.float32,
        )


def kernel(hidden_states, gate_weight, gate_up_proj, down_proj):
    return pl.pallas_call(
        _moe_body,
        grid=(E,),
        in_specs=[
            pl.BlockSpec((T, D_MODEL), lambda e: (0, 0)),
            pl.BlockSpec((E, D_MODEL), lambda e: (0, 0)),
            pl.BlockSpec((1, 2 * D_FF, D_MODEL), lambda e: (e, 0, 0)),
            pl.BlockSpec((1, D_MODEL, D_FF), lambda e: (e, 0, 0)),
        ],
        out_specs=pl.BlockSpec((T, D_MODEL), lambda e: (0, 0)),
        out_shape=jax.ShapeDtypeStruct((T, D_MODEL), jnp.float32),
        scratch_shapes=[
            pltpu.VMEM((E, T), jnp.float32),
            pltpu.VMEM((E, T), jnp.float32),
            pltpu.VMEM((E, T), jnp.float32),
            pltpu.VMEM((T, D_MODEL), jnp.float32),
            pltpu.VMEM((E * BM, D_MODEL), jnp.float32),
            pltpu.VMEM((8, T), jnp.float32),
        ],
        compiler_params=pltpu.CompilerParams(
            dimension_semantics=("arbitrary",),
        ),
    )(hidden_states, gate_weight, gate_up_proj, down_proj)
